# baseline (speedup 1.0000x reference)
import functools

import jax
import jax.numpy as jnp
from jax import lax
from jax.experimental import pallas as pl
from jax.experimental.pallas import tpu as pltpu
from jax.experimental.pallas import tpu_sc as plsc

NC = 2
NS = 16
NW = NC * NS
BLK = 80
NBUF = 4


def _support_body(x_ref, w_ref, out_ref):
    out_ref[...] = jnp.dot(x_ref[...], w_ref[...],
                           preferred_element_type=jnp.float32)


def _support_matmul(x, w):
    n, d = x.shape
    return pl.pallas_call(
        _support_body,
        out_shape=jax.ShapeDtypeStruct((n, d), jnp.float32),
    )(x, w)


def _make_seg_sum(n, d, e):
    ept = e // NW
    nmain = ept // BLK
    assert ept * NW == e and nmain * BLK == ept and BLK % 8 == 0
    rows_pt = (n // NS) // 8 * 8
    rows_rem = n - rows_pt * NS
    assert rows_rem >= 0 and rows_rem % 8 == 0
    mesh = plsc.VectorSubcoreMesh(core_axis_name="c", subcore_axis_name="s")

    @functools.partial(
        pl.kernel,
        out_type=jax.ShapeDtypeStruct((NC, n, d), jnp.float32),
        mesh=mesh,
        scratch_types=[
            pltpu.VMEM_SHARED((n, d), jnp.float32),
            pltpu.VMEM((NBUF, BLK), jnp.int32),
            pltpu.VMEM((NBUF, BLK), jnp.float32),
            pltpu.VMEM((NBUF, BLK), jnp.int32),
            pltpu.VMEM((NBUF, BLK, d), jnp.float32),
            pltpu.SemaphoreType.DMA((NBUF,)),
            pltpu.SemaphoreType.DMA((NBUF,)),
            pltpu.SemaphoreType.DMA((NBUF,)),
            pltpu.SemaphoreType.DMA((NBUF,)),
            pltpu.SemaphoreType.DMA((NBUF,)),
        ],
    )
    def seg(support_hbm, src_hbm, dst_hbm, w_hbm,
            out_hbm, acc, src_r, w_r, dst_r, rows,
            gsem, ssem, dsem, wsem, xsem):
        cid = lax.axis_index("c")
        sid = lax.axis_index("s")
        wid = cid * NS + sid
        base_m = pl.multiple_of(wid * (nmain * BLK), 8)

        zero16 = jnp.zeros((16,), jnp.float32)

        def zrow(r, c):
            for k in range(d // 16):
                rows[0, r, pl.ds(k * 16, 16)] = zero16
            return c

        lax.fori_loop(0, BLK, zrow, 0)
        off = pl.multiple_of(sid * rows_pt, 8)
        for z in range(rows_pt // BLK):
            pltpu.sync_copy(rows.at[0],
                            acc.at[pl.ds(off + z * BLK, BLK)])
        zrem = rows_pt - (rows_pt // BLK) * BLK
        if zrem:
            pltpu.sync_copy(rows.at[0, pl.ds(0, zrem)],
                            acc.at[pl.ds(off + (rows_pt // BLK) * BLK, zrem)])
        if rows_rem:
            @pl.when(sid == 0)
            def _():
                pltpu.sync_copy(rows.at[0, pl.ds(0, rows_rem)],
                                acc.at[pl.ds(rows_pt * NS, rows_rem)])

        def x_desc(b, j):
            return pltpu.make_async_copy(
                src_hbm.at[pl.ds(base_m + b * BLK, BLK)], src_r.at[j],
                xsem.at[j])

        def g_desc(b, j):
            return pltpu.make_async_copy(
                support_hbm.at[src_r.at[j]], rows.at[j], gsem.at[j])

        def d_desc(b, j):
            return pltpu.make_async_copy(
                dst_hbm.at[pl.ds(base_m + b * BLK, BLK)], dst_r.at[j],
                dsem.at[j])

        def w_desc(b, j):
            return pltpu.make_async_copy(
                w_hbm.at[pl.ds(base_m + b * BLK, BLK)], w_r.at[j],
                wsem.at[j])

        def s_desc(b, j):
            return pltpu.make_async_copy(
                rows.at[j], acc.at[dst_r.at[j]], ssem.at[j])

        def mult(buf_ref, w_ref, nrow):
            @plsc.parallel_loop(0, nrow // 16)
            def _(t):
                wv = w_ref[pl.ds(t * 16, 16)]
                for jj in range(16):
                    wj = wv[jj]
                    row = t * 16 + jj
                    for k in range(d // 16):
                        sl = pl.ds(k * 16, 16)
                        buf_ref[row, sl] = buf_ref[row, sl] * wj

        for k in range(NBUF):
            x_desc(k, k).start()
        plsc.subcore_barrier()
        for b in range(2):
            x_desc(b, b).wait()
            d_desc(b, b).start()
            w_desc(b, b).start()
            g_desc(b, b).start()

        def step(b, j):
            g_desc(b, j).wait()
            d_desc(b, j).wait()
            w_desc(b, j).wait()

            @pl.when(b + NBUF < nmain)
            def _():
                x_desc(b + NBUF, j).start()

            jn = (j + 2) % NBUF

            @pl.when(b >= 2)
            def _():
                s_desc(b - 2, jn).wait()

            @pl.when(b + 2 < nmain)
            def _():
                x_desc(b + 2, jn).wait()
                d_desc(b + 2, jn).start()
                w_desc(b + 2, jn).start()
                g_desc(b + 2, jn).start()

            mult(rows.at[j], w_r.at[j], BLK)
            s_desc(b, j).start(add=True)

        nquad = nmain // NBUF

        def quad(q, c):
            for j in range(NBUF):
                step(q * NBUF + j, j)
            return c

        lax.fori_loop(0, nquad, quad, 0)
        for b in range(nquad * NBUF, nmain):
            step(b, b % NBUF)
        s_desc(nmain - 2, (nmain - 2) % NBUF).wait()
        s_desc(nmain - 1, (nmain - 1) % NBUF).wait()

        plsc.subcore_barrier()
        pltpu.sync_copy(
            acc.at[pl.ds(off, rows_pt)],
            out_hbm.at[cid, pl.ds(off, rows_pt)],
        )
        if rows_rem:
            @pl.when(sid == 0)
            def _():
                pltpu.sync_copy(
                    acc.at[pl.ds(rows_pt * NS, rows_rem)],
                    out_hbm.at[cid, pl.ds(rows_pt * NS, rows_rem)],
                )

    return seg


def _loss_body(p_ref, lbl_ref, w_ref, b_ref, out_ref):
    i = pl.program_id(0)
    ng = pl.num_programs(0)
    n_total = p_ref.shape[1] * ng
    emb = p_ref[0] + p_ref[1]
    logits = jnp.dot(emb, w_ref[...], preferred_element_type=jnp.float32)
    logits = logits + b_ref[...]
    m = jnp.max(logits, axis=1, keepdims=True)
    ex = jnp.exp(logits - m)
    lse = jnp.log(jnp.sum(ex, axis=1, keepdims=True)) + m
    col = lax.broadcasted_iota(jnp.int32, logits.shape, 1)
    oh = col == lbl_ref[...]
    part = jnp.sum(jnp.where(oh, logits, 0.0)) - jnp.sum(lse)

    @pl.when(i == 0)
    def _():
        out_ref[0, 0] = 0.0

    out_ref[0, 0] += part

    @pl.when(i == ng - 1)
    def _():
        out_ref[0, 0] = -out_ref[0, 0] / n_total


def _loss(partials, labels2d, w_disc, b2):
    _, n, d = partials.shape
    nparts = w_disc.shape[1]
    r = 2000
    grid = n // r
    assert grid * r == n
    return pl.pallas_call(
        _loss_body,
        grid=(grid,),
        in_specs=[
            pl.BlockSpec((NC, r, d), lambda i: (0, i, 0)),
            pl.BlockSpec((r, 1), lambda i: (i, 0)),
            pl.BlockSpec((d, nparts), lambda i: (0, 0)),
            pl.BlockSpec((1, nparts), lambda i: (0, 0)),
        ],
        out_specs=pl.BlockSpec((1, 1), lambda i: (0, 0),
                               memory_space=pltpu.SMEM),
        out_shape=jax.ShapeDtypeStruct((1, 1), jnp.float32),
    )(partials, labels2d, w_disc, b2)


def kernel(encoder_features, adj_weight, W_gcn, W_disc, b_disc, edge_index,
           pseudo_labels):
    n, d = encoder_features.shape
    e = edge_index.shape[1]

    support = _support_matmul(encoder_features, W_gcn)

    seg = _make_seg_sum(n, d, e)
    partials = seg(support, edge_index[0], edge_index[1], adj_weight)

    labels2d = pseudo_labels.astype(jnp.int32)[:, None]
    acc = _loss(partials, labels2d, W_disc, b_disc[None, :])
    return acc[0, 0]


# speedup vs baseline: 1.0818x; 1.0818x over previous
import functools

import jax
import jax.numpy as jnp
from jax import lax
from jax.experimental import pallas as pl
from jax.experimental.pallas import tpu as pltpu
from jax.experimental.pallas import tpu_sc as plsc

NC = 2
NS = 16
NW = NC * NS
BLK = 80
NBUF = 4


def _support_body(x_ref, w_ref, out_ref):
    out_ref[...] = jnp.dot(x_ref[...], w_ref[...],
                           preferred_element_type=jnp.float32)


def _support_matmul(x, w):
    n, d = x.shape
    return pl.pallas_call(
        _support_body,
        out_shape=jax.ShapeDtypeStruct((n, d), jnp.float32),
    )(x, w)


def _make_seg_sum(n, d, e):
    ept = e // NW
    nmain = ept // BLK
    assert ept * NW == e and nmain * BLK == ept and BLK % 8 == 0
    rows_pt = (n // NS) // 8 * 8
    rows_rem = n - rows_pt * NS
    assert rows_rem >= 0 and rows_rem % 8 == 0
    mesh = plsc.VectorSubcoreMesh(core_axis_name="c", subcore_axis_name="s")

    @functools.partial(
        pl.kernel,
        out_type=jax.ShapeDtypeStruct((NC, n, d), jnp.float32),
        mesh=mesh,
        scratch_types=[
            pltpu.VMEM_SHARED((n, d), jnp.float32),
            pltpu.VMEM((NBUF, BLK), jnp.int32),
            pltpu.VMEM((NBUF, BLK), jnp.float32),
            pltpu.VMEM((NBUF, BLK), jnp.int32),
            pltpu.VMEM((NBUF, BLK, d), jnp.float32),
            pltpu.SemaphoreType.DMA((NBUF,)),
            pltpu.SemaphoreType.DMA((NBUF,)),
            pltpu.SemaphoreType.DMA((NBUF,)),
            pltpu.SemaphoreType.DMA((NBUF,)),
            pltpu.SemaphoreType.DMA((NBUF,)),
        ],
    )
    def seg(support_hbm, src_hbm, dst_hbm, w_hbm,
            out_hbm, acc, src_r, w_r, dst_r, rows,
            gsem, ssem, dsem, wsem, xsem):
        cid = lax.axis_index("c")
        sid = lax.axis_index("s")
        wid = cid * NS + sid
        base_m = pl.multiple_of(wid * (nmain * BLK), 8)

        zero16 = jnp.zeros((16,), jnp.float32)

        def zrow(r, c):
            for k in range(d // 16):
                rows[0, r, pl.ds(k * 16, 16)] = zero16
            return c

        lax.fori_loop(0, BLK, zrow, 0)
        off = pl.multiple_of(sid * rows_pt, 8)
        for z in range(rows_pt // BLK):
            pltpu.sync_copy(rows.at[0],
                            acc.at[pl.ds(off + z * BLK, BLK)])
        zrem = rows_pt - (rows_pt // BLK) * BLK
        if zrem:
            pltpu.sync_copy(rows.at[0, pl.ds(0, zrem)],
                            acc.at[pl.ds(off + (rows_pt // BLK) * BLK, zrem)])
        if rows_rem:
            @pl.when(sid == 0)
            def _():
                pltpu.sync_copy(rows.at[0, pl.ds(0, rows_rem)],
                                acc.at[pl.ds(rows_pt * NS, rows_rem)])

        def x_desc(b, j):
            return pltpu.make_async_copy(
                src_hbm.at[pl.ds(base_m + b * BLK, BLK)], src_r.at[j],
                xsem.at[j])

        def g_desc(b, j):
            return pltpu.make_async_copy(
                support_hbm.at[src_r.at[j]], rows.at[j], gsem.at[j])

        def d_desc(b, j):
            return pltpu.make_async_copy(
                dst_hbm.at[pl.ds(base_m + b * BLK, BLK)], dst_r.at[j],
                dsem.at[j])

        def w_desc(b, j):
            return pltpu.make_async_copy(
                w_hbm.at[pl.ds(base_m + b * BLK, BLK)], w_r.at[j],
                wsem.at[j])

        def s_desc(b, j):
            return pltpu.make_async_copy(
                rows.at[j], acc.at[dst_r.at[j]], ssem.at[j])

        def mult(buf_ref, w_ref, nrow):
            def grp(t, c2):
                wv = w_ref[pl.ds(t * 16, 16)]
                for jj in range(16):
                    wj = wv[jj]
                    row = t * 16 + jj
                    for k in range(d // 16):
                        sl = pl.ds(k * 16, 16)
                        buf_ref[row, sl] = buf_ref[row, sl] * wj
                return c2

            lax.fori_loop(0, nrow // 16, grp, 0)

        for k in range(NBUF):
            x_desc(k, k).start()
        plsc.subcore_barrier()
        for b in range(2):
            x_desc(b, b).wait()
            d_desc(b, b).start()
            w_desc(b, b).start()
            g_desc(b, b).start()

        def step(b, j):
            g_desc(b, j).wait()
            d_desc(b, j).wait()
            w_desc(b, j).wait()

            @pl.when(b + NBUF < nmain)
            def _():
                x_desc(b + NBUF, j).start()

            jn = (j + 2) % NBUF

            @pl.when(b >= 2)
            def _():
                s_desc(b - 2, jn).wait()

            @pl.when(b + 2 < nmain)
            def _():
                x_desc(b + 2, jn).wait()
                d_desc(b + 2, jn).start()
                w_desc(b + 2, jn).start()
                g_desc(b + 2, jn).start()

            mult(rows.at[j], w_r.at[j], BLK)
            s_desc(b, j).start(add=True)

        nquad = nmain // NBUF

        def quad(q, c):
            for j in range(NBUF):
                step(q * NBUF + j, j)
            return c

        lax.fori_loop(0, nquad, quad, 0)
        for b in range(nquad * NBUF, nmain):
            step(b, b % NBUF)
        s_desc(nmain - 2, (nmain - 2) % NBUF).wait()
        s_desc(nmain - 1, (nmain - 1) % NBUF).wait()

        plsc.subcore_barrier()
        pltpu.sync_copy(
            acc.at[pl.ds(off, rows_pt)],
            out_hbm.at[cid, pl.ds(off, rows_pt)],
        )
        if rows_rem:
            @pl.when(sid == 0)
            def _():
                pltpu.sync_copy(
                    acc.at[pl.ds(rows_pt * NS, rows_rem)],
                    out_hbm.at[cid, pl.ds(rows_pt * NS, rows_rem)],
                )

    return seg


def _loss_body(p_ref, lbl_ref, w_ref, b_ref, out_ref):
    i = pl.program_id(0)
    ng = pl.num_programs(0)
    n_total = p_ref.shape[1] * ng
    emb = p_ref[0] + p_ref[1]
    logits = jnp.dot(emb, w_ref[...], preferred_element_type=jnp.float32)
    logits = logits + b_ref[...]
    m = jnp.max(logits, axis=1, keepdims=True)
    ex = jnp.exp(logits - m)
    lse = jnp.log(jnp.sum(ex, axis=1, keepdims=True)) + m
    col = lax.broadcasted_iota(jnp.int32, logits.shape, 1)
    oh = col == lbl_ref[...]
    part = jnp.sum(jnp.where(oh, logits, 0.0)) - jnp.sum(lse)

    @pl.when(i == 0)
    def _():
        out_ref[0, 0] = 0.0

    out_ref[0, 0] += part

    @pl.when(i == ng - 1)
    def _():
        out_ref[0, 0] = -out_ref[0, 0] / n_total


def _loss(partials, labels2d, w_disc, b2):
    _, n, d = partials.shape
    nparts = w_disc.shape[1]
    r = 2000
    grid = n // r
    assert grid * r == n
    return pl.pallas_call(
        _loss_body,
        grid=(grid,),
        in_specs=[
            pl.BlockSpec((NC, r, d), lambda i: (0, i, 0)),
            pl.BlockSpec((r, 1), lambda i: (i, 0)),
            pl.BlockSpec((d, nparts), lambda i: (0, 0)),
            pl.BlockSpec((1, nparts), lambda i: (0, 0)),
        ],
        out_specs=pl.BlockSpec((1, 1), lambda i: (0, 0),
                               memory_space=pltpu.SMEM),
        out_shape=jax.ShapeDtypeStruct((1, 1), jnp.float32),
    )(partials, labels2d, w_disc, b2)


def kernel(encoder_features, adj_weight, W_gcn, W_disc, b_disc, edge_index,
           pseudo_labels):
    n, d = encoder_features.shape
    e = edge_index.shape[1]

    support = _support_matmul(encoder_features, W_gcn)

    seg = _make_seg_sum(n, d, e)
    partials = seg(support, edge_index[0], edge_index[1], adj_weight)

    labels2d = pseudo_labels.astype(jnp.int32)[:, None]
    acc = _loss(partials, labels2d, W_disc, b_disc[None, :])
    return acc[0, 0]
